# Initial kernel scaffold; baseline (speedup 1.0000x reference)
import functools

import jax
import jax.numpy as jnp
from jax import lax
from jax.experimental import pallas as pl
from jax.experimental.pallas import tpu as pltpu
from jax.experimental.pallas import tpu_sc as plsc

VOCAB = 1_000_000
D_MODEL = 64
BATCH = 16384
SEQ = 50

NC = 2
NS = 16
NW = NC * NS
N_IDX = BATCH * SEQ
CHUNK = 128
N_CHUNKS = N_IDX // CHUNK
CPW = N_CHUNKS // NW
SLOTS = 8
NGROUPS = CPW // SLOTS


def _sc_body(ids_hbm, table_hbm, out_hbm, idx_v, rows_v, gsem, wsem):
    wid = lax.axis_index("s") * NC + lax.axis_index("c")
    chunk0 = wid * CPW

    pltpu.sync_copy(ids_hbm.at[pl.ds(chunk0, CPW)], idx_v)

    def gather(j_local, b):
        pltpu.async_copy(
            table_hbm.at[idx_v.at[j_local]], rows_v.at[b], gsem.at[b]
        )

    def gather_wait(b):
        pltpu.make_async_copy(
            table_hbm.at[idx_v.at[0]], rows_v.at[b], gsem.at[b]
        ).wait()

    def write(j_local, b):
        off = (chunk0 + j_local) * CHUNK
        pltpu.async_copy(rows_v.at[b], out_hbm.at[pl.ds(off, CHUNK)], wsem.at[b])

    def write_wait(b):
        pltpu.make_async_copy(
            rows_v.at[b], out_hbm.at[pl.ds(0, CHUNK)], wsem.at[b]
        ).wait()

    for b in range(SLOTS):
        gather(b, b)

    @pl.loop(0, NGROUPS)
    def _grp(g):
        base = g * SLOTS
        for b in range(SLOTS):
            gather_wait(b)
            write(base + b, b)
        for b in range(SLOTS):
            write_wait(b)

            @pl.when(g < NGROUPS - 1)
            def _():
                gather(base + SLOTS + b, b)


def _sc_lookup(ids2d, table):
    mesh = plsc.VectorSubcoreMesh(
        core_axis_name="c", subcore_axis_name="s", num_cores=NC, num_subcores=NS
    )
    fn = pl.kernel(
        _sc_body,
        out_type=jax.ShapeDtypeStruct((N_IDX, D_MODEL), jnp.float32),
        mesh=mesh,
        scratch_types=[
            pltpu.VMEM((CPW, CHUNK), jnp.int32),
            pltpu.VMEM((SLOTS, CHUNK, D_MODEL), jnp.float32),
            pltpu.SemaphoreType.DMA((SLOTS,)),
            pltpu.SemaphoreType.DMA((SLOTS,)),
        ],
    )
    return fn(ids2d, table)


def kernel(input_ids, delta_table):
    ids2d = jnp.reshape(input_ids.astype(jnp.int32), (N_CHUNKS, CHUNK))
    out = _sc_lookup(ids2d, delta_table)
    return jnp.reshape(out, (BATCH, SEQ, D_MODEL))


# speedup vs baseline: 1.8718x; 1.8718x over previous
import functools

import jax
import jax.numpy as jnp
from jax import lax
from jax.experimental import pallas as pl
from jax.experimental.pallas import tpu as pltpu
from jax.experimental.pallas import tpu_sc as plsc

VOCAB = 1_000_000
D_MODEL = 64
BATCH = 16384
SEQ = 50

NC = 2
NS = 16
NW = NC * NS
N_IDX = BATCH * SEQ
CHUNK = 128
N_CHUNKS = N_IDX // CHUNK
CPW = N_CHUNKS // NW
SLOTS = 8
NGROUPS = CPW // SLOTS


def _sc_body(ids_hbm, table_hbm, out_hbm, idx_v, rows_v, gsem, wsem):
    wid = lax.axis_index("s") * NC + lax.axis_index("c")
    chunk0 = wid * CPW

    pltpu.sync_copy(ids_hbm.at[pl.ds(chunk0, CPW)], idx_v)

    def gather(j_local, b):
        pltpu.async_copy(
            table_hbm.at[idx_v.at[j_local]], rows_v.at[b], gsem.at[b]
        )

    def gather_wait(b):
        pltpu.make_async_copy(
            table_hbm.at[idx_v.at[0]], rows_v.at[b], gsem.at[b]
        ).wait()

    def write(j_local, b):
        off = (chunk0 + j_local) * CHUNK
        pltpu.async_copy(rows_v.at[b], out_hbm.at[pl.ds(off, CHUNK)], wsem.at[b])

    def write_wait(b):
        pltpu.make_async_copy(
            rows_v.at[b], out_hbm.at[pl.ds(0, CHUNK)], wsem.at[b]
        ).wait()

    for b in range(SLOTS):
        gather(b, b)

    @pl.loop(0, NGROUPS)
    def _grp(g):
        base = g * SLOTS
        for b in range(SLOTS):
            gather_wait(b)
            write(base + b, b)
        for b in range(SLOTS):
            write_wait(b)

            @pl.when(g < NGROUPS - 1)
            def _():
                gather(base + SLOTS + b, b)


def _sc_lookup(ids2d, table):
    mesh = plsc.VectorSubcoreMesh(
        core_axis_name="c", subcore_axis_name="s", num_cores=NC, num_subcores=NS
    )
    fn = pl.kernel(
        _sc_body,
        out_type=jax.ShapeDtypeStruct((N_IDX, D_MODEL), jnp.float32),
        mesh=mesh,
        scratch_types=[
            pltpu.VMEM((CPW, CHUNK), jnp.int32),
            pltpu.VMEM((SLOTS, CHUNK, D_MODEL), jnp.float32),
            pltpu.SemaphoreType.DMA((SLOTS,)),
            pltpu.SemaphoreType.DMA((SLOTS,)),
        ],
        compiler_params=pltpu.CompilerParams(use_tc_tiling_on_sc=False),
    )
    return fn(ids2d, table)


def kernel(input_ids, delta_table):
    ids2d = jnp.reshape(input_ids.astype(jnp.int32), (N_CHUNKS, CHUNK))
    out = _sc_lookup(ids2d, delta_table)
    return jnp.reshape(out, (BATCH, SEQ, D_MODEL))
